# Initial kernel scaffold; baseline (speedup 1.0000x reference)
import jax
import jax.numpy as jnp
from jax.experimental import pallas as pl


def kernel(node_attr, coords, batch_id, perturb_mask, edges, edge_type_attr, We1, be1, We2, be2, L0_We1, L0_be1, L0_We2, L0_be2, L0_A, L1_We1, L1_be1, L1_We2, L1_be2, L1_A, Wf1, bf1, Wf2, bf2):
    raise NotImplementedError("write your pallas kernel here")


# speedup vs baseline: 90.6475x; 90.6475x over previous
import functools

import jax
import jax.numpy as jnp
import numpy as np
from jax import lax
from jax.experimental import pallas as pl
from jax.experimental.pallas import tpu as pltpu
from jax.experimental.pallas import tpu_sc as plsc

_F32 = jnp.float32
_L = 128


def _mesh():
    return plsc.VectorSubcoreMesh(core_axis_name="c", subcore_axis_name="s")


def _gather_pair(table, src, dst):
    E = src.shape[0]
    CHR = 2
    CH = CHR * _L
    NCHUNK = E // CH
    NW = 32
    NIT = (NCHUNK + NW - 1) // NW

    @functools.partial(
        pl.kernel,
        out_type=(
            jax.ShapeDtypeStruct((E, 128), _F32),
            jax.ShapeDtypeStruct((E, 128), _F32),
        ),
        mesh=_mesh(),
        scratch_types=(
            [pltpu.VMEM((_L,), jnp.int32) for _ in range(2 * CHR)]
            + [pltpu.VMEM((CH, 128), _F32), pltpu.VMEM((CH, 128), _F32),
               pltpu.SemaphoreType.DMA]
        ),
    )
    def k(src_h, dst_h, t_h, xs_h, xd_h, *scr):
        si = scr[:CHR]
        di = scr[CHR:2 * CHR]
        xsv, xdv, sem = scr[2 * CHR:]
        wid = lax.axis_index("s") * 2 + lax.axis_index("c")

        def body(i, carry):
            chunk = wid + NW * i

            @pl.when(chunk < NCHUNK)
            def _():
                e0 = chunk * CH
                ds = []
                for j in range(CHR):
                    ds.append(pltpu.async_copy(
                        src_h.at[pl.ds(e0 + j * _L, _L)], si[j], sem))
                    ds.append(pltpu.async_copy(
                        dst_h.at[pl.ds(e0 + j * _L, _L)], di[j], sem))
                for d in ds:
                    d.wait()
                ds = []
                for j in range(CHR):
                    sl = pl.ds(j * _L, _L)
                    ds.append(pltpu.async_copy(t_h.at[si[j]], xsv.at[sl], sem))
                    ds.append(pltpu.async_copy(t_h.at[di[j]], xdv.at[sl], sem))
                for d in ds:
                    d.wait()
                pltpu.sync_copy(xsv, xs_h.at[pl.ds(e0, CH)])
                pltpu.sync_copy(xdv, xd_h.at[pl.ds(e0, CH)])

            return carry

        lax.fori_loop(0, NIT, body, 0)

    return k(src, dst, table)


def _scatter_add(msg_slabs, dst, zer, n_nodes):
    NCH = len(msg_slabs)
    E = dst.shape[0]
    CH = _L
    NCHUNK = E // CH
    NT = 16
    NIT = (NCHUNK + NT - 1) // NT
    RA = (-(-n_nodes // NT) + 7) // 8 * 8
    RB = n_nodes - (NT - 1) * RA

    @functools.partial(
        pl.kernel,
        out_type=tuple(jax.ShapeDtypeStruct((n_nodes, 32), _F32)
                       for _ in range(NCH)),
        mesh=_mesh(),
        scratch_types=[
            pltpu.VMEM((CH,), jnp.int32),
            pltpu.VMEM((CH, 32), _F32),
            pltpu.VMEM_SHARED((n_nodes, 32), _F32),
            pltpu.SemaphoreType.DMA,
        ],
        compiler_params=pltpu.CompilerParams(use_tc_tiling_on_sc=False),
    )
    def k(*refs):
        msg_h = refs[:NCH]
        dst_h = refs[NCH]
        z_h = refs[NCH + 1]
        out_h = refs[NCH + 2:2 * NCH + 2]
        dj, mv, acc, sem = refs[2 * NCH + 2:]

        c = lax.axis_index("c")
        s = lax.axis_index("s")
        for q in range(NCH):
            mine = (q % 2) == c

            @pl.when(mine & (s < NT - 1))
            def _():
                pltpu.sync_copy(z_h.at[pl.ds(s * RA, RA)],
                                acc.at[pl.ds(s * RA, RA)])

            @pl.when(mine & (s == NT - 1))
            def _():
                pltpu.sync_copy(z_h.at[pl.ds((NT - 1) * RA, RB)],
                                acc.at[pl.ds((NT - 1) * RA, RB)])

            plsc.subcore_barrier()

            @pl.when(mine)
            def _():
                def body(i, carry):
                    chunk = s + NT * i

                    @pl.when(chunk < NCHUNK)
                    def _():
                        e0 = chunk * CH
                        d1 = pltpu.async_copy(
                            msg_h[q].at[pl.ds(e0, CH)], mv, sem)
                        d2 = pltpu.async_copy(
                            dst_h.at[pl.ds(e0, CH)], dj, sem)
                        d1.wait()
                        d2.wait()
                        pltpu.sync_copy(mv, acc.at[dj], add=True)

                    return carry

                lax.fori_loop(0, NIT, body, 0)

            plsc.subcore_barrier()

            @pl.when(mine & (s < NT - 1))
            def _():
                pltpu.sync_copy(acc.at[pl.ds(s * RA, RA)],
                                out_h[q].at[pl.ds(s * RA, RA)])

            @pl.when(mine & (s == NT - 1))
            def _():
                pltpu.sync_copy(acc.at[pl.ds((NT - 1) * RA, RB)],
                                out_h[q].at[pl.ds((NT - 1) * RA, RB)])

            plsc.subcore_barrier()

    return k(*msg_slabs, dst, zer)


def _edge_tc(ev8, eta, xs, xd, F,
             We1, be1, We2, be2, W1e, W1t, W1s, W1d, b1, W2, b2, Ax, As):
    first = ev8 is None
    E = xs.shape[0]
    Dop = W2.shape[1]
    NCH = Dop // 32
    B = 2000
    grid = (E // B,)

    def body(*refs):
        i = 0
        if not first:
            ev8_r = refs[0]
            i = 1
        (eta_r, xs_r, xd_r, We1_r, be1_r, We2_r, be2_r,
         W1e_r, W1t_r, W1s_r, W1d_r, b1_r, W2_r, b2_r, Ax_r, As_r) = \
            refs[i:i + 16]
        outs = refs[i + 16:]
        msg_rs = outs[:NCH]
        xs_v = xs_r[...]
        xd_v = xd_r[...]
        if first:
            ev = xd_v[:, 48:56] - xs_v[:, 48:56]
            outs[NCH][...] = ev
        else:
            ev = ev8_r[...]
        d2 = jnp.sum(ev * ev, axis=1, keepdims=True) + 1e-12
        n = jnp.sqrt(d2)
        inv = 1.0 / n
        step = 20.0 / 31.0
        off = lax.broadcasted_iota(jnp.int32, (1, 32), 1).astype(_F32) * step
        g = jnp.exp((-0.5 / (step * step)) * (n - off) ** 2)
        ee = jnp.maximum(g @ We1_r[...] + be1_r[...], 0.0) @ We2_r[...] \
            + be2_r[...]
        v = ev * inv
        x = v[:, 0:1]; y = v[:, 1:2]; z = v[:, 2:3]
        s3 = float(np.sqrt(3.0)); s15 = float(np.sqrt(15.0))
        s5 = float(np.sqrt(5.0))
        sh = jnp.concatenate([
            jnp.ones_like(x), s3 * x, s3 * y, s3 * z,
            s15 * x * y, s15 * y * z, (s5 / 2.0) * (3.0 * z * z - 1.0),
            s15 * x * z, (s15 / 2.0) * (x * x - y * y),
            jnp.zeros((B, 7), _F32),
        ], axis=1)
        xs_f = xs_v[:, :F]
        h = ee @ W1e_r[...] + eta_r[...] @ W1t_r[...] \
            + xs_v[:, :48] @ W1s_r[...] + xd_v[:, :48] @ W1d_r[...] + b1_r[...]
        h = jnp.maximum(h, 0.0)
        w = h @ W2_r[...] + b2_r[...]
        msg = (xs_f @ Ax_r[...] + sh @ As_r[...]) * w
        for q in range(NCH):
            msg_rs[q][...] = msg[:, q * 32:(q + 1) * 32]

    def eb(d):
        return pl.BlockSpec((B, d), lambda i: (i, 0))

    def wb(shape):
        return pl.BlockSpec(shape, lambda i: (0,) * len(shape))

    in_specs = ([] if first else [eb(8)]) + [
        eb(32), eb(128), eb(128),
        wb(We1.shape), wb(be1.shape), wb(We2.shape), wb(be2.shape),
        wb(W1e.shape), wb(W1t.shape), wb(W1s.shape), wb(W1d.shape),
        wb(b1.shape), wb(W2.shape), wb(b2.shape),
        wb(Ax.shape), wb(As.shape),
    ]
    out_specs = [eb(32) for _ in range(NCH)] + ([eb(8)] if first else [])
    out_shape = [jax.ShapeDtypeStruct((E, 32), _F32) for _ in range(NCH)] \
        + ([jax.ShapeDtypeStruct((E, 8), _F32)] if first else [])
    args = ([] if first else [ev8]) + [eta, xs, xd, We1, be1, We2, be2,
                                       W1e, W1t, W1s, W1d, b1, W2, b2, Ax, As]
    res = pl.pallas_call(
        body, grid=grid, in_specs=in_specs,
        out_specs=out_specs, out_shape=out_shape,
    )(*args)
    if first:
        return tuple(res[:NCH]), res[NCH]
    return tuple(res), None


def _node_update0(agg_slabs, nattr):
    N = nattr.shape[0]
    Bn = 1000
    grid = (N // Bn,)

    def body(*refs):
        agg_rs = refs[:4]
        na_r = refs[4]
        x1_r, x1a_r = refs[5], refs[6]
        u = jnp.concatenate([r[...] for r in agg_rs], axis=1) * 0.25
        m = jnp.mean(u, axis=1, keepdims=True)
        v = jnp.mean((u - m) ** 2, axis=1, keepdims=True)
        upd = (u - m) * lax.rsqrt(v + 1e-5)
        x1 = jnp.concatenate(
            [na_r[...], jnp.zeros((Bn, 80), _F32)], axis=1) + upd
        x1_r[...] = x1
        x1a_r[...] = x1[:, :48]

    def eb(d):
        return pl.BlockSpec((Bn, d), lambda i: (i, 0))

    return pl.pallas_call(
        body,
        grid=grid,
        in_specs=[eb(32)] * 4 + [eb(48)],
        out_specs=[eb(128), eb(48)],
        out_shape=[jax.ShapeDtypeStruct((N, 128), _F32),
                   jax.ShapeDtypeStruct((N, 48), _F32)],
    )(*agg_slabs, nattr)


def _node_update1(agg_slabs, x1a, Wf1, bf1, Wf2, bf2):
    N = x1a.shape[0]
    Bn = 1000
    grid = (N // Bn,)

    def body(*refs):
        agg_rs = refs[:7]
        x1a_r = refs[7]
        Wf1_r, bf1_r, Wf2_r, bf2_r = refs[8:12]
        out_r = refs[12]
        u = jnp.concatenate([r[...] for r in agg_rs], axis=1)[:, :208] * 0.25
        m = jnp.mean(u, axis=1, keepdims=True)
        v = jnp.mean((u - m) ** 2, axis=1, keepdims=True)
        upd = (u - m) * lax.rsqrt(v + 1e-5)
        ne = x1a_r[...] + upd[:, :48]
        h = jnp.maximum(ne @ Wf1_r[...] + bf1_r[...], 0.0)
        out_r[...] = h @ Wf2_r[...] + bf2_r[...]

    def eb(d):
        return pl.BlockSpec((Bn, d), lambda i: (i, 0))

    def wb(shape):
        return pl.BlockSpec(shape, lambda i: (0,) * len(shape))

    return pl.pallas_call(
        body,
        grid=grid,
        in_specs=[eb(32)] * 7 + [eb(48),
                                 wb(Wf1.shape), wb(bf1.shape),
                                 wb(Wf2.shape), wb(bf2.shape)],
        out_specs=eb(48),
        out_shape=jax.ShapeDtypeStruct((N, 48), _F32),
    )(*agg_slabs, x1a, Wf1, bf1, Wf2, bf2)


def kernel(node_attr, coords, batch_id, perturb_mask, edges, edge_type_attr,
           We1, be1, We2, be2,
           L0_We1, L0_be1, L0_We2, L0_be2, L0_A,
           L1_We1, L1_be1, L1_We2, L1_be2, L1_A,
           Wf1, bf1, Wf2, bf2):
    N, ns = node_attr.shape
    E = edges.shape[1]

    src = edges[0].astype(jnp.int32)
    dst = edges[1].astype(jnp.int32)
    table0 = jnp.concatenate(
        [node_attr, coords, jnp.zeros((N, 77), _F32)], axis=1)
    zer = jnp.zeros((N, 32), _F32)

    def r1(b):
        return b.reshape(1, -1)

    W1e0, W1t0 = L0_We1[0:32], L0_We1[32:64]
    W1s0, W1d0 = L0_We1[64:112], L0_We1[112:160]
    A0x = L0_A[:48]
    A0s = jnp.pad(L0_A[48:57], ((0, 7), (0, 0)))

    W1e1, W1t1 = L1_We1[0:32], L1_We1[32:64]
    W1s1, W1d1 = L1_We1[64:112], L1_We1[112:160]
    W2_1 = jnp.pad(L1_We2, ((0, 0), (0, 16)))
    b2_1 = jnp.pad(L1_be2, ((0, 16),))
    A1x = jnp.pad(L1_A[:128], ((0, 0), (0, 16)))
    A1s = jnp.pad(L1_A[128:137], ((0, 7), (0, 16)))

    xs0, xd0 = _gather_pair(table0, src, dst)
    msg0, ev8 = _edge_tc(None, edge_type_attr, xs0, xd0, 48,
                         We1, r1(be1), We2, r1(be2),
                         W1e0, W1t0, W1s0, W1d0, r1(L0_be1),
                         L0_We2, r1(L0_be2), A0x, A0s)
    agg0 = _scatter_add(msg0, dst, zer, N)
    x1, x1a = _node_update0(agg0, node_attr)

    xs1, xd1 = _gather_pair(x1, src, dst)
    msg1, _ = _edge_tc(ev8, edge_type_attr, xs1, xd1, 128,
                       We1, r1(be1), We2, r1(be2),
                       W1e1, W1t1, W1s1, W1d1, r1(L1_be1),
                       W2_1, r1(b2_1), A1x, A1s)
    agg1 = _scatter_add(msg1, dst, zer, N)
    return _node_update1(agg1, x1a, Wf1, r1(bf1), Wf2, r1(bf2))
